# Initial kernel scaffold; baseline (speedup 1.0000x reference)
import jax
import jax.numpy as jnp
from jax.experimental import pallas as pl


def kernel(xyz, params):
    raise NotImplementedError("write your pallas kernel here")


# speedup vs baseline: 12.3785x; 12.3785x over previous
import functools

import jax
import jax.numpy as jnp
import numpy as np
from jax import lax
from jax.experimental import pallas as pl
from jax.experimental.pallas import tpu as pltpu
from jax.experimental.pallas import tpu_sc as plsc

F32 = jnp.float32
I32 = jnp.int32


def _fps_body(S, N, xyz_ref, out_ref, dist_ref):
    B = xyz_ref.shape[0]
    X = xyz_ref[:, 0, :]
    Y = xyz_ref[:, 1, :]
    Z = xyz_ref[:, 2, :]
    iota = lax.broadcasted_iota(I32, (B, N), 1)
    dist_ref[:] = jnp.full((B, N), 1e10, F32)

    def body(i, far):
        m = iota == far
        cx = jnp.sum(jnp.where(m, X, 0.0), 1, keepdims=True)
        cy = jnp.sum(jnp.where(m, Y, 0.0), 1, keepdims=True)
        cz = jnp.sum(jnp.where(m, Z, 0.0), 1, keepdims=True)
        d = (X - cx) ** 2 + (Y - cy) ** 2 + (Z - cz) ** 2
        nd = jnp.minimum(dist_ref[:], d)
        dist_ref[:] = nd
        mx = jnp.max(nd, 1, keepdims=True)
        nf = jnp.min(jnp.where(nd == mx, iota, N), 1, keepdims=True)
        cen = jnp.concatenate([cx, cy, cz], 1)
        out_ref[pl.ds(i, 1)] = cen[None]
        return nf

    lax.fori_loop(0, S, body, jnp.zeros((B, 1), I32))


def _fps(xyz_t, S):
    B, _, N = xyz_t.shape
    return pl.pallas_call(
        functools.partial(_fps_body, S, N),
        out_shape=jax.ShapeDtypeStruct((S, B, 3), F32),
        scratch_shapes=[pltpu.VMEM((B, N), F32)],
    )(xyz_t)


def _sel_body(N, S, ns, r2, cT_ref, pT_ref, prow_ref, out_ref):
    b = pl.program_id(0)
    P = pT_ref[0]
    cT = cT_ref[0]
    pr = prow_ref[0]
    dotm = lax.dot_general(pr, cT, (((1,), (0,)), ((), ())),
                           preferred_element_type=F32)
    asq = jnp.sum(cT * cT, 0, keepdims=True)
    bsq = jnp.sum(pr * pr, 1, keepdims=True)
    d = (-2.0 * dotm + asq) + bsq
    mf = (d <= r2).astype(F32)

    NB = N // 128
    li = lax.broadcasted_iota(I32, (128, 128), 0)
    lj = lax.broadcasted_iota(I32, (128, 128), 1)
    Linc = (li >= lj).astype(F32)
    cins = []
    tots = []
    for j in range(NB):
        mj = mf[j * 128:(j + 1) * 128]
        cj = lax.dot_general(Linc, mj, (((1,), (0,)), ((), ())),
                             preferred_element_type=F32)
        cins.append(cj)
        tots.append(cj[127:128])
    tot = jnp.concatenate(tots, 0)
    bi = lax.broadcasted_iota(I32, (NB, NB), 0)
    bj = lax.broadcasted_iota(I32, (NB, NB), 1)
    TLs = (bj < bi).astype(F32)
    base = lax.dot_general(TLs, tot, (((1,), (0,)), ((), ())),
                           preferred_element_type=F32)
    cum = jnp.concatenate(
        [cins[j] + base[j:j + 1] for j in range(NB)], 0)
    count = base[NB - 1:NB] + tot[NB - 1:NB]

    def kbody(k, _):
        kf = k.astype(F32)
        ck = jnp.sum((cum <= kf).astype(F32), 0, keepdims=True)
        out_ref[0, pl.ds(k, 1), :] = ck.astype(I32)
        return 0

    lax.fori_loop(0, ns, kbody, 0)
    v = out_ref[0]
    kcol = lax.broadcasted_iota(I32, (ns, 1), 0)
    ci = count.astype(I32)
    v = jnp.where(kcol < ci, v, v[0:1])
    v = jnp.minimum(v, N - 1)
    out_ref[0] = v + b * N


def _ball_query(cen_bt, xyz_t, xyz_rows, r2, ns):
    B, _, S = cen_bt.shape
    N = xyz_t.shape[2]
    return pl.pallas_call(
        functools.partial(_sel_body, N, S, ns, r2),
        grid=(B,),
        in_specs=[
            pl.BlockSpec((1, 3, S), lambda b: (b, 0, 0)),
            pl.BlockSpec((1, 3, N), lambda b: (b, 0, 0)),
            pl.BlockSpec((1, N, 3), lambda b: (b, 0, 0)),
        ],
        out_specs=pl.BlockSpec((1, ns, S), lambda b: (b, 0, 0)),
        out_shape=jax.ShapeDtypeStruct((B, ns, S), I32),
    )(cen_bt, xyz_t, xyz_rows)


_SC_NC = 2
_SC_NS = 16
_SC_NW = _SC_NC * _SC_NS


def _gather_rows(table, idx):
    V, D = table.shape
    Btot = idx.shape[0]
    b_per_w = Btot // _SC_NW
    CH = 128
    nch = b_per_w // CH
    idx2 = idx.reshape(Btot // CH, CH)
    mesh = plsc.VectorSubcoreMesh(core_axis_name="c", subcore_axis_name="s")

    @functools.partial(
        pl.kernel, mesh=mesh,
        out_type=jax.ShapeDtypeStruct((Btot, D), F32),
        compiler_params=pltpu.CompilerParams(use_tc_tiling_on_sc=False),
        scratch_types=[
            pltpu.VMEM((nch, CH), I32),
            pltpu.VMEM((CH, D), F32),
            pltpu.SemaphoreType.DMA,
        ],
    )
    def k(table_hbm, idx_hbm, out_hbm, idx_v, rows_v, sem):
        wid = lax.axis_index("s") * _SC_NC + lax.axis_index("c")
        base = wid * b_per_w
        pltpu.sync_copy(idx_hbm.at[pl.ds(wid * nch, nch)], idx_v)

        def body(c, _):
            pltpu.async_copy(table_hbm.at[idx_v.at[c]], rows_v, sem).wait()
            pltpu.sync_copy(rows_v, out_hbm.at[pl.ds(base + c * CH, CH)])
            return 0

        lax.fori_loop(0, nch, body, 0, unroll=False)

    return k(table, idx2)


def _l1_body(Ntot, g_ref, cen_ref, w_ref, b_ref, y_ref, s_ref):
    b = pl.program_id(0)
    rows = g_ref[0]
    g = rows[:, 0:3] - cen_ref[0]
    y = lax.dot_general(g, w_ref[:], (((1,), (1,)), ((), ())),
                        preferred_element_type=F32) + b_ref[:]
    y_ref[0] = y

    @pl.when(b == 0)
    def _():
        s_ref[:] = jnp.zeros_like(s_ref)

    s_ref[0:1] += jnp.sum(y, 0, keepdims=True)
    s_ref[1:2] += jnp.sum(y * y, 0, keepdims=True)


def _first_layer(gath, cen_exp, W, bias, Ntot):
    B, M, Din = gath.shape
    Cout = W.shape[0]
    return pl.pallas_call(
        functools.partial(_l1_body, Ntot),
        grid=(B,),
        in_specs=[
            pl.BlockSpec((1, M, Din), lambda b: (b, 0, 0)),
            pl.BlockSpec((1, M, 3), lambda b: (b, 0, 0)),
            pl.BlockSpec(W.shape, lambda b: (0, 0)),
            pl.BlockSpec((1, Cout), lambda b: (0, 0)),
        ],
        out_specs=[
            pl.BlockSpec((1, M, Cout), lambda b: (b, 0, 0)),
            pl.BlockSpec((2, Cout), lambda b: (0, 0)),
        ],
        out_shape=[
            jax.ShapeDtypeStruct((B, M, Cout), F32),
            jax.ShapeDtypeStruct((2, Cout), F32),
        ],
    )(gath, cen_exp, W, bias)


def _cat_body(Ntot, g_ref, cen_ref, w_ref, b_ref, y_ref, s_ref):
    b = pl.program_id(0)
    rows = g_ref[0]
    g = rows - cen_ref[0]
    y = lax.dot_general(g, w_ref[:], (((1,), (1,)), ((), ())),
                        preferred_element_type=F32) + b_ref[:]
    y_ref[0] = y

    @pl.when(b == 0)
    def _():
        s_ref[:] = jnp.zeros_like(s_ref)

    s_ref[0:1] += jnp.sum(y, 0, keepdims=True)
    s_ref[1:2] += jnp.sum(y * y, 0, keepdims=True)


def _first_layer_cat(gath, cen_pad_exp, Wp, bias, Ntot):
    B, M, Din = gath.shape
    Cout = Wp.shape[0]
    return pl.pallas_call(
        functools.partial(_cat_body, Ntot),
        grid=(B,),
        in_specs=[
            pl.BlockSpec((1, M, Din), lambda b: (b, 0, 0)),
            pl.BlockSpec((1, M, Din), lambda b: (b, 0, 0)),
            pl.BlockSpec(Wp.shape, lambda b: (0, 0)),
            pl.BlockSpec((1, Cout), lambda b: (0, 0)),
        ],
        out_specs=[
            pl.BlockSpec((1, M, Cout), lambda b: (b, 0, 0)),
            pl.BlockSpec((2, Cout), lambda b: (0, 0)),
        ],
        out_shape=[
            jax.ShapeDtypeStruct((B, M, Cout), F32),
            jax.ShapeDtypeStruct((2, Cout), F32),
        ],
    )(gath, cen_pad_exp, Wp, bias)


def _plain_first_body(g_ref, w_ref, b_ref, y_ref, s_ref):
    b = pl.program_id(0)
    y = lax.dot_general(g_ref[0], w_ref[:], (((1,), (1,)), ((), ())),
                        preferred_element_type=F32) + b_ref[:]
    y_ref[0] = y

    @pl.when(b == 0)
    def _():
        s_ref[:] = jnp.zeros_like(s_ref)

    s_ref[0:1] += jnp.sum(y, 0, keepdims=True)
    s_ref[1:2] += jnp.sum(y * y, 0, keepdims=True)


def _plain_first(x, W, bias):
    B, M, Din = x.shape
    Cout = W.shape[0]
    return pl.pallas_call(
        _plain_first_body,
        grid=(B,),
        in_specs=[
            pl.BlockSpec((1, M, Din), lambda b: (b, 0, 0)),
            pl.BlockSpec(W.shape, lambda b: (0, 0)),
            pl.BlockSpec((1, Cout), lambda b: (0, 0)),
        ],
        out_specs=[
            pl.BlockSpec((1, M, Cout), lambda b: (b, 0, 0)),
            pl.BlockSpec((2, Cout), lambda b: (0, 0)),
        ],
        out_shape=[
            jax.ShapeDtypeStruct((B, M, Cout), F32),
            jax.ShapeDtypeStruct((2, Cout), F32),
        ],
    )(x, W, bias)


def _bn(y, s_ref, Ntot, gamma_ref, beta_ref):
    n = jnp.float32(Ntot)
    m = s_ref[0:1] / n
    var = s_ref[1:2] / n
    return gamma_ref[:] * (y - m) / jnp.sqrt(var + 1e-5) + beta_ref[:]


def _ssd_body(Ntot, y_ref, s_in_ref, o_ref):
    b = pl.program_id(0)
    m = s_in_ref[0:1] / jnp.float32(Ntot)
    d = y_ref[0] - m

    @pl.when(b == 0)
    def _():
        o_ref[:] = jnp.zeros_like(o_ref)

    o_ref[0:1] += jnp.sum(d * d, 0, keepdims=True)


def _stats2(y, sums, Ntot):
    B, M, C = y.shape
    ssd = pl.pallas_call(
        functools.partial(_ssd_body, Ntot),
        grid=(B,),
        in_specs=[
            pl.BlockSpec((1, M, C), lambda b: (b, 0, 0)),
            pl.BlockSpec((2, C), lambda b: (0, 0)),
        ],
        out_specs=pl.BlockSpec((1, C), lambda b: (0, 0)),
        out_shape=jax.ShapeDtypeStruct((1, C), F32),
    )(y, sums)
    return jnp.concatenate([sums[0:1], ssd], 0)


def _mid_body(Ntot, y_ref, s_in_ref, g_ref, be_ref, w_ref, b_ref,
              o_ref, s_ref):
    b = pl.program_id(0)
    z = jnp.maximum(_bn(y_ref[0], s_in_ref, Ntot, g_ref, be_ref), 0.0)
    y = lax.dot_general(z, w_ref[:], (((1,), (1,)), ((), ())),
                        preferred_element_type=F32) + b_ref[:]
    o_ref[0] = y

    @pl.when(b == 0)
    def _():
        s_ref[:] = jnp.zeros_like(s_ref)

    s_ref[0:1] += jnp.sum(y, 0, keepdims=True)
    s_ref[1:2] += jnp.sum(y * y, 0, keepdims=True)


def _mid_layer(y_prev, stats, gamma, beta, W, bias, Ntot):
    B, M, Cin = y_prev.shape
    Cout = W.shape[0]
    return pl.pallas_call(
        functools.partial(_mid_body, Ntot),
        grid=(B,),
        in_specs=[
            pl.BlockSpec((1, M, Cin), lambda b: (b, 0, 0)),
            pl.BlockSpec((2, Cin), lambda b: (0, 0)),
            pl.BlockSpec((1, Cin), lambda b: (0, 0)),
            pl.BlockSpec((1, Cin), lambda b: (0, 0)),
            pl.BlockSpec(W.shape, lambda b: (0, 0)),
            pl.BlockSpec((1, Cout), lambda b: (0, 0)),
        ],
        out_specs=[
            pl.BlockSpec((1, M, Cout), lambda b: (b, 0, 0)),
            pl.BlockSpec((2, Cout), lambda b: (0, 0)),
        ],
        out_shape=[
            jax.ShapeDtypeStruct((B, M, Cout), F32),
            jax.ShapeDtypeStruct((2, Cout), F32),
        ],
    )(y_prev, stats, gamma, beta, W, bias)


def _max_body(Ntot, ns, S, y_ref, s_in_ref, g_ref, be_ref, o_ref):
    z = jnp.maximum(_bn(y_ref[0], s_in_ref, Ntot, g_ref, be_ref), 0.0)
    m = z[0:S]
    for k in range(1, ns):
        m = jnp.maximum(m, z[k * S:(k + 1) * S])
    o_ref[0] = m


def _bn_relu_max(y, stats, gamma, beta, ns, S, Ntot):
    B, M, C = y.shape
    return pl.pallas_call(
        functools.partial(_max_body, Ntot, ns, S),
        grid=(B,),
        in_specs=[
            pl.BlockSpec((1, M, C), lambda b: (b, 0, 0)),
            pl.BlockSpec((2, C), lambda b: (0, 0)),
            pl.BlockSpec((1, C), lambda b: (0, 0)),
            pl.BlockSpec((1, C), lambda b: (0, 0)),
        ],
        out_specs=pl.BlockSpec((1, S, C), lambda b: (b, 0, 0)),
        out_shape=jax.ShapeDtypeStruct((B, S, C), F32),
    )(y, stats, gamma, beta)


def _last_max_body(Ntot, y_ref, s_in_ref, g_ref, be_ref, w_ref, b_ref, o_ref):
    z = jnp.maximum(_bn(y_ref[0], s_in_ref, Ntot, g_ref, be_ref), 0.0)
    y = lax.dot_general(z, w_ref[:], (((1,), (1,)), ((), ())),
                        preferred_element_type=F32) + b_ref[:]
    o_ref[0] = jnp.max(y, 0, keepdims=True)


def _last_layer_max(y_prev, stats, gamma, beta, W, bias, Ntot):
    B, M, Cin = y_prev.shape
    Cout = W.shape[0]
    return pl.pallas_call(
        functools.partial(_last_max_body, Ntot),
        grid=(B,),
        in_specs=[
            pl.BlockSpec((1, M, Cin), lambda b: (b, 0, 0)),
            pl.BlockSpec((2, Cin), lambda b: (0, 0)),
            pl.BlockSpec((1, Cin), lambda b: (0, 0)),
            pl.BlockSpec((1, Cin), lambda b: (0, 0)),
            pl.BlockSpec(W.shape, lambda b: (0, 0)),
            pl.BlockSpec((1, Cout), lambda b: (0, 0)),
        ],
        out_specs=pl.BlockSpec((1, 1, Cout), lambda b: (b, 0, 0)),
        out_shape=jax.ShapeDtypeStruct((B, 1, Cout), F32),
    )(y_prev, stats, gamma, beta, W, bias).reshape(B, Cout)


def _fc_body(w1_ref, b1_ref, g1_ref, be1_ref, w2_ref, b2_ref, g2_ref,
             be2_ref, x_ref, o_ref):
    def fc(x, w, bb, g, be):
        y = lax.dot_general(x, w[:], (((1,), (1,)), ((), ())),
                            preferred_element_type=F32) + bb[:]
        m = jnp.mean(y, 0, keepdims=True)
        var = jnp.mean((y - m) ** 2, 0, keepdims=True)
        z = g[:] * (y - m) / jnp.sqrt(var + 1e-5) + be[:]
        return jnp.maximum(z, 0.0)

    h = fc(x_ref[:], w1_ref, b1_ref, g1_ref, be1_ref)
    o_ref[:] = fc(h, w2_ref, b2_ref, g2_ref, be2_ref)


def _fc_head(x, fc1, fc2):
    W1, b1, g1, be1 = fc1
    W2, b2, g2, be2 = fc2
    args = [W1, b1.reshape(1, -1), g1.reshape(1, -1), be1.reshape(1, -1),
            W2, b2.reshape(1, -1), g2.reshape(1, -1), be2.reshape(1, -1), x]
    return pl.pallas_call(
        _fc_body,
        out_shape=jax.ShapeDtypeStruct((x.shape[0], W2.shape[0]), F32),
    )(*args)


def _sa_mlp(gathered, cen_exp, layers, ns, S, cat_mode):
    B, M, _ = gathered.shape
    Ntot = B * M
    (W1, b1, g1, be1), (W2, b2, g2, be2), (W3, b3, g3, be3) = layers
    if cat_mode:
        Wp = jnp.pad(W1, ((0, 0), (0, gathered.shape[2] - W1.shape[1])))
        y1, s1 = _first_layer_cat(gathered, cen_exp, Wp, b1.reshape(1, -1),
                                  Ntot)
    else:
        y1, s1 = _first_layer(gathered, cen_exp, W1, b1.reshape(1, -1), Ntot)
    st1 = _stats2(y1, s1, Ntot)
    y2, s2 = _mid_layer(y1, st1, g1.reshape(1, -1), be1.reshape(1, -1),
                        W2, b2.reshape(1, -1), Ntot)
    st2 = _stats2(y2, s2, Ntot)
    y3, s3 = _mid_layer(y2, st2, g2.reshape(1, -1), be2.reshape(1, -1),
                        W3, b3.reshape(1, -1), Ntot)
    st3 = _stats2(y3, s3, Ntot)
    return _bn_relu_max(y3, st3, g3.reshape(1, -1), be3.reshape(1, -1),
                        ns, S, Ntot)


def kernel(xyz, params):
    B, _, N = xyz.shape
    S1, ns1, r1 = 512, 32, 0.2
    S2, ns2, r2 = 128, 64, 0.4

    xyz_rows = jnp.transpose(xyz, (0, 2, 1))

    cen1_sbc = _fps(xyz, S1)
    new_xyz1 = jnp.transpose(cen1_sbc, (1, 0, 2))
    cen1_t = jnp.transpose(cen1_sbc, (1, 2, 0))
    idx1 = _ball_query(cen1_t, xyz, xyz_rows, np.float32(r1 ** 2), ns1)
    table1 = jnp.pad(xyz_rows.reshape(B * N, 3), ((0, 0), (0, 13)))
    g1 = _gather_rows(table1, idx1.reshape(-1))
    g1 = g1.reshape(B, ns1 * S1, 16)
    cen1_exp = jnp.broadcast_to(
        new_xyz1[:, None], (B, ns1, S1, 3)).reshape(B, ns1 * S1, 3)
    l1_points = _sa_mlp(g1, cen1_exp, params['sa1'], ns1, S1, False)

    cen2_sbc = _fps(cen1_t, S2)
    new_xyz2 = jnp.transpose(cen2_sbc, (1, 0, 2))
    cen2_t = jnp.transpose(cen2_sbc, (1, 2, 0))
    idx2 = _ball_query(cen2_t, cen1_t, new_xyz1, np.float32(r2 ** 2), ns2)
    feat1 = jnp.concatenate([new_xyz1, l1_points], -1)
    table2 = jnp.pad(feat1.reshape(B * S1, 131), ((0, 0), (0, 13)))
    g2 = _gather_rows(table2, idx2.reshape(-1))
    g2 = g2.reshape(B, ns2 * S2, 144)
    cen2_pad = jnp.pad(new_xyz2, ((0, 0), (0, 0), (0, 141)))
    cen2_exp = jnp.broadcast_to(
        cen2_pad[:, None], (B, ns2, S2, 144)).reshape(B, ns2 * S2, 144)
    l2_points = _sa_mlp(g2, cen2_exp, params['sa2'], ns2, S2, True)

    g3 = jnp.concatenate([new_xyz2, l2_points], -1)
    (W1, b1, g1p, be1), (W2, b2, g2p, be2), (W3, b3, _, _) = params['sa3']
    y1, s1 = _plain_first(g3, W1, b1.reshape(1, -1))
    st1 = _stats2(y1, s1, B * S2)
    y2, s2 = _mid_layer(y1, st1, g1p.reshape(1, -1), be1.reshape(1, -1),
                        W2, b2.reshape(1, -1), B * S2)
    st2 = _stats2(y2, s2, B * S2)
    feat = _last_layer_max(y2, st2, g2p.reshape(1, -1), be2.reshape(1, -1),
                           W3, b3.reshape(1, -1), B * S2)

    return _fc_head(feat, params['fc1'], params['fc2'])
